# baseline (speedup 1.0000x reference)
import functools

import jax
import jax.numpy as jnp
import numpy as np
from jax.experimental import pallas as pl
from jax.experimental.pallas import tpu as pltpu

NUM_GROUPS = 4
NUM_EMBED = 1024
EMBED_DIM = 384
COMMIT = 0.25
CODE_DIM = EMBED_DIM // NUM_GROUPS
B = 8
H = 32
W = 32
HW = H * W
TOTAL_ROWS = B * HW


def _vq_body(x_ref, e_ref, et_ref, zq_ref, loss_ref, kld_ref, perp_ref,
             hist_ref, sse_ref, loss_acc_ref, perp_acc_ref):
    g = pl.program_id(0)
    b = pl.program_id(1)

    @pl.when(b == 0)
    def _init_group():
        hist_ref[...] = jnp.zeros_like(hist_ref)
        sse_ref[...] = jnp.zeros_like(sse_ref)

    @pl.when((b == 0) & (g == 0))
    def _init_all():
        loss_acc_ref[...] = jnp.zeros_like(loss_acc_ref)
        perp_acc_ref[...] = jnp.zeros_like(perp_acc_ref)

    x = x_ref[0].reshape(CODE_DIM, HW)
    e = e_ref[0]
    et = et_ref[0]

    s = jnp.dot(e, x, preferred_element_type=jnp.float32)
    e2 = jnp.sum(e * e, axis=1, keepdims=True)
    x2 = jnp.sum(x * x, axis=0, keepdims=True)
    d = (x2 + e2) - 2.0 * s

    idx = jnp.argmin(d, axis=0).reshape(1, HW)
    dmin = jnp.min(d, axis=0, keepdims=True)

    codes = jax.lax.broadcasted_iota(jnp.int32, (NUM_EMBED, HW), 0)
    r = (codes == idx).astype(jnp.float32)

    zq = jnp.dot(et, r, preferred_element_type=jnp.float32)
    zq_ref[0] = zq.reshape(CODE_DIM, H, W)

    hist_ref[...] += jnp.sum(r, axis=1, keepdims=True)
    sse_ref[...] += jnp.sum(dmin, keepdims=True)

    @pl.when(b == B - 1)
    def _group_final():
        probs = hist_ref[...] / float(TOTAL_ROWS)
        ent = -jnp.sum(probs * jnp.log(probs + 1e-10), keepdims=True)
        perp_acc_ref[...] += jnp.exp(ent)
        loss_acc_ref[...] += ((1.0 + COMMIT) * sse_ref[...]
                              / float(B * HW * CODE_DIM))

    @pl.when((b == B - 1) & (g == NUM_GROUPS - 1))
    def _final():
        loss_ref[...] = loss_acc_ref[...] / float(NUM_GROUPS)
        perp_ref[...] = perp_acc_ref[...] / float(NUM_GROUPS)
        kld_ref[...] = jnp.full_like(
            kld_ref, np.log(float(NUM_EMBED)) * float(HW) * NUM_GROUPS)


@functools.partial(jax.jit, static_argnames=("interpret",))
def _vq_call(inputs, embeds, embeds_t, interpret=False):
    grid = (NUM_GROUPS, B)
    out = pl.pallas_call(
        _vq_body,
        grid=grid,
        in_specs=[
            pl.BlockSpec((1, CODE_DIM, H, W), lambda g, b: (b, g, 0, 0)),
            pl.BlockSpec((1, NUM_EMBED, CODE_DIM), lambda g, b: (g, 0, 0)),
            pl.BlockSpec((1, CODE_DIM, NUM_EMBED), lambda g, b: (g, 0, 0)),
        ],
        out_specs=[
            pl.BlockSpec((1, CODE_DIM, H, W), lambda g, b: (b, g, 0, 0)),
            pl.BlockSpec((1, 1), lambda g, b: (0, 0)),
            pl.BlockSpec((B, 1), lambda g, b: (0, 0)),
            pl.BlockSpec((1, 1), lambda g, b: (0, 0)),
        ],
        out_shape=[
            jax.ShapeDtypeStruct((B, EMBED_DIM, H, W), jnp.float32),
            jax.ShapeDtypeStruct((1, 1), jnp.float32),
            jax.ShapeDtypeStruct((B, 1), jnp.float32),
            jax.ShapeDtypeStruct((1, 1), jnp.float32),
        ],
        scratch_shapes=[
            pltpu.VMEM((NUM_EMBED, 1), jnp.float32),
            pltpu.VMEM((1, 1), jnp.float32),
            pltpu.VMEM((1, 1), jnp.float32),
            pltpu.VMEM((1, 1), jnp.float32),
        ],
        compiler_params=pltpu.CompilerParams(
            dimension_semantics=("arbitrary", "arbitrary")),
        interpret=interpret,
    )(inputs, embeds, embeds_t)
    return out


def kernel(inputs, embeds, interpret=False):
    embeds_t = jnp.swapaxes(embeds, 1, 2)
    z_q, loss, kldiv_r, perp = _vq_call(inputs, embeds, embeds_t,
                                        interpret=interpret)
    return z_q, loss.reshape(()), kldiv_r, perp.reshape(())


# speedup vs baseline: 1.2603x; 1.2603x over previous
import functools

import jax
import jax.numpy as jnp
import numpy as np
from jax.experimental import pallas as pl
from jax.experimental.pallas import tpu as pltpu

NUM_GROUPS = 4
NUM_EMBED = 1024
EMBED_DIM = 384
COMMIT = 0.25
CODE_DIM = EMBED_DIM // NUM_GROUPS
B = 8
HW = 1024
KAUG = CODE_DIM + 8
TOTAL_ROWS = B * HW


def _vq_body(x_ref, e_ref, et_ref, zq_ref, loss_ref, kld_ref, perp_ref,
             xa_ref, ea_ref, racc_ref, sse_ref, loss_acc_ref, perp_acc_ref):
    g = pl.program_id(0)
    b = pl.program_id(1)

    @pl.when((b == 0) & (g == 0))
    def _init_all():
        loss_acc_ref[...] = jnp.zeros_like(loss_acc_ref)
        perp_acc_ref[...] = jnp.zeros_like(perp_acc_ref)
        row = jax.lax.broadcasted_iota(jnp.int32, (8, HW), 0)
        xa_ref[CODE_DIM:KAUG, :] = jnp.where(row == 0, 1.0, 0.0)

    @pl.when(b == 0)
    def _init_group():
        e = e_ref[0]
        e2 = jnp.sum(e * e, axis=1, keepdims=True)
        ea_ref[...] = jnp.concatenate(
            [-2.0 * e, e2, jnp.zeros((NUM_EMBED, KAUG - CODE_DIM - 1),
                                     jnp.float32)], axis=1)
        racc_ref[...] = jnp.zeros_like(racc_ref)
        sse_ref[...] = jnp.zeros_like(sse_ref)

    x = x_ref[0, 0]
    xa_ref[0:CODE_DIM, :] = x

    d = jnp.dot(ea_ref[...], xa_ref[...],
                preferred_element_type=jnp.float32)

    idx = jnp.argmin(d, axis=0).reshape(1, HW)
    dmin = jnp.min(d, axis=0, keepdims=True)

    codes = jax.lax.broadcasted_iota(jnp.int32, (NUM_EMBED, HW), 0)
    r = (codes == idx).astype(jnp.bfloat16)

    zq_ref[0, 0] = jnp.dot(et_ref[0], r,
                           preferred_element_type=jnp.float32)

    racc_ref[...] += r
    sse_ref[...] += (jnp.sum(dmin, keepdims=True)
                     + jnp.sum(x * x, keepdims=True))

    @pl.when(b == B - 1)
    def _group_final():
        hist = jnp.sum(racc_ref[...].astype(jnp.float32), axis=1,
                       keepdims=True)
        probs = hist / float(TOTAL_ROWS)
        ent = -jnp.sum(probs * jnp.log(probs + 1e-10), keepdims=True)
        perp_acc_ref[...] += jnp.exp(ent)
        loss_acc_ref[...] += ((1.0 + COMMIT) * sse_ref[...]
                              / float(B * HW * CODE_DIM))

    @pl.when((b == B - 1) & (g == NUM_GROUPS - 1))
    def _final():
        loss_ref[...] = loss_acc_ref[...] / float(NUM_GROUPS)
        perp_ref[...] = perp_acc_ref[...] / float(NUM_GROUPS)
        kld_ref[...] = jnp.full_like(
            kld_ref, np.log(float(NUM_EMBED)) * float(HW) * NUM_GROUPS)


@functools.partial(jax.jit, static_argnames=("interpret",))
def _vq_call(x4, embeds, embeds_t_bf16, interpret=False):
    grid = (NUM_GROUPS, B)
    out = pl.pallas_call(
        _vq_body,
        grid=grid,
        in_specs=[
            pl.BlockSpec((1, 1, CODE_DIM, HW), lambda g, b: (b, g, 0, 0)),
            pl.BlockSpec((1, NUM_EMBED, CODE_DIM), lambda g, b: (g, 0, 0)),
            pl.BlockSpec((1, CODE_DIM, NUM_EMBED), lambda g, b: (g, 0, 0)),
        ],
        out_specs=[
            pl.BlockSpec((1, 1, CODE_DIM, HW), lambda g, b: (b, g, 0, 0)),
            pl.BlockSpec((1, 1), lambda g, b: (0, 0)),
            pl.BlockSpec((B, 1), lambda g, b: (0, 0)),
            pl.BlockSpec((1, 1), lambda g, b: (0, 0)),
        ],
        out_shape=[
            jax.ShapeDtypeStruct((B, NUM_GROUPS, CODE_DIM, HW), jnp.float32),
            jax.ShapeDtypeStruct((1, 1), jnp.float32),
            jax.ShapeDtypeStruct((B, 1), jnp.float32),
            jax.ShapeDtypeStruct((1, 1), jnp.float32),
        ],
        scratch_shapes=[
            pltpu.VMEM((KAUG, HW), jnp.float32),
            pltpu.VMEM((NUM_EMBED, KAUG), jnp.float32),
            pltpu.VMEM((NUM_EMBED, HW), jnp.bfloat16),
            pltpu.VMEM((1, 1), jnp.float32),
            pltpu.VMEM((1, 1), jnp.float32),
            pltpu.VMEM((1, 1), jnp.float32),
        ],
        compiler_params=pltpu.CompilerParams(
            dimension_semantics=("arbitrary", "arbitrary")),
        interpret=interpret,
    )(x4, embeds, embeds_t_bf16)
    return out


def kernel(inputs, embeds, interpret=False):
    x4 = inputs.reshape(B, NUM_GROUPS, CODE_DIM, HW)
    embeds_t_bf16 = jnp.swapaxes(embeds, 1, 2).astype(jnp.bfloat16)
    zq4, loss, kldiv_r, perp = _vq_call(x4, embeds, embeds_t_bf16,
                                        interpret=interpret)
    z_q = zq4.reshape(B, EMBED_DIM, 32, 32)
    return z_q, loss.reshape(()), kldiv_r, perp.reshape(())
